# baseline (speedup 1.0000x reference)
import functools

import jax
import jax.numpy as jnp
from jax import lax
from jax.experimental import pallas as pl
from jax.experimental.pallas import tpu as pltpu
from jax.experimental.pallas import tpu_sc as plsc

NUM_CORES = 2
NUM_SUBCORES = 16
LANES = 16
NW = NUM_CORES * NUM_SUBCORES

EMBED_DIM = 64
CHUNK = 128


def _body(src_hbm, tgt_hbm, emb_hbm, gat_hbm, out_hbm,
          sidx, tidx, wt, ws, gt, gs, out_v, sem, *, b_per_w):
    wid = lax.axis_index("s") * NUM_CORES + lax.axis_index("c")
    base = wid * b_per_w

    pltpu.sync_copy(src_hbm.at[pl.ds(base, b_per_w)], sidx)
    pltpu.sync_copy(tgt_hbm.at[pl.ds(base, b_per_w)], tidx)

    lane = lax.iota(jnp.int32, LANES)
    n_chunks = b_per_w // CHUNK
    n_slices = EMBED_DIM // LANES

    for c in range(n_chunks):
        tsl = tidx.at[pl.ds(c * CHUNK, CHUNK)]
        ssl = sidx.at[pl.ds(c * CHUNK, CHUNK)]
        cps = [
            pltpu.async_copy(emb_hbm.at[tsl], wt, sem),
            pltpu.async_copy(emb_hbm.at[ssl], ws, sem),
            pltpu.async_copy(gat_hbm.at[tsl], gt, sem),
            pltpu.async_copy(gat_hbm.at[ssl], gs, sem),
        ]
        for cp in cps:
            cp.wait()

        def group_body(g, _, c=c):
            res = jnp.zeros((LANES,), jnp.float32)
            for j in range(LANES):
                row = g * LANES + j
                acc = jnp.zeros((LANES,), jnp.float32)
                for k in range(n_slices):
                    sl = pl.ds(k * LANES, LANES)
                    grad = wt[row, sl] - ws[row, sl]
                    gsum = gt[row, sl] + gs[row, sl]
                    denom = 1.0 + jnp.exp(-gsum)
                    acc = acc + grad / denom
                s = jnp.sum(acc)
                res = jnp.where(lane == j, s, res)
            out_v[pl.ds(c * CHUNK + g * LANES, LANES)] = res
            return 0

        lax.fori_loop(0, CHUNK // LANES, group_body, 0)

    pltpu.sync_copy(out_v, out_hbm.at[pl.ds(base, b_per_w)])


@jax.jit
def kernel(source_nodes, target_nodes, node_embeddings, gates):
    batch = source_nodes.shape[0]
    b_per_w = batch // NW
    mesh = plsc.VectorSubcoreMesh(
        core_axis_name="c", subcore_axis_name="s",
        num_cores=NUM_CORES, num_subcores=NUM_SUBCORES)
    run = pl.kernel(
        functools.partial(_body, b_per_w=b_per_w),
        out_type=jax.ShapeDtypeStruct((batch,), jnp.float32),
        mesh=mesh,
        compiler_params=pltpu.CompilerParams(
            needs_layout_passes=False, use_tc_tiling_on_sc=False),
        scratch_types=[
            pltpu.VMEM((b_per_w,), jnp.int32),
            pltpu.VMEM((b_per_w,), jnp.int32),
            pltpu.VMEM((CHUNK, EMBED_DIM), jnp.float32),
            pltpu.VMEM((CHUNK, EMBED_DIM), jnp.float32),
            pltpu.VMEM((CHUNK, EMBED_DIM), jnp.float32),
            pltpu.VMEM((CHUNK, EMBED_DIM), jnp.float32),
            pltpu.VMEM((b_per_w,), jnp.float32),
            pltpu.SemaphoreType.DMA,
        ],
    )
    return run(source_nodes.astype(jnp.int32), target_nodes.astype(jnp.int32),
               node_embeddings, gates)


# speedup vs baseline: 1.0012x; 1.0012x over previous
import functools

import jax
import jax.numpy as jnp
from jax import lax
from jax.experimental import pallas as pl
from jax.experimental.pallas import tpu as pltpu
from jax.experimental.pallas import tpu_sc as plsc

NUM_CORES = 2
NUM_SUBCORES = 16
LANES = 16
NW = NUM_CORES * NUM_SUBCORES

EMBED_DIM = 64
PAIR = 2 * EMBED_DIM
CHUNK = 128


def _body(src_hbm, tgt_hbm, emb_hbm, gat_hbm, out_hbm,
          sidx, tidx, spar, tpar, wt, ws, gt, gs, out_v, sem, *, b_per_w):
    wid = lax.axis_index("s") * NUM_CORES + lax.axis_index("c")
    base = wid * b_per_w

    pltpu.sync_copy(src_hbm.at[pl.ds(base, b_per_w)], sidx)
    pltpu.sync_copy(tgt_hbm.at[pl.ds(base, b_per_w)], tidx)

    n_vec = b_per_w // LANES
    for v in range(n_vec):
        sl = pl.ds(v * LANES, LANES)
        si = sidx[sl]
        ti = tidx[sl]
        spar[sl] = lax.shift_left(lax.bitwise_and(si, 1), 6)
        tpar[sl] = lax.shift_left(lax.bitwise_and(ti, 1), 6)
        sidx[sl] = lax.shift_right_logical(si, 1)
        tidx[sl] = lax.shift_right_logical(ti, 1)

    lane = lax.iota(jnp.int32, LANES)
    n_chunks = b_per_w // CHUNK
    n_slices = EMBED_DIM // LANES

    for c in range(n_chunks):
        tsl = tidx.at[pl.ds(c * CHUNK, CHUNK)]
        ssl = sidx.at[pl.ds(c * CHUNK, CHUNK)]
        cps = [
            pltpu.async_copy(emb_hbm.at[tsl], wt, sem),
            pltpu.async_copy(emb_hbm.at[ssl], ws, sem),
            pltpu.async_copy(gat_hbm.at[tsl], gt, sem),
            pltpu.async_copy(gat_hbm.at[ssl], gs, sem),
        ]
        for cp in cps:
            cp.wait()

        def group_body(g, _, c=c):
            gbase = g * LANES
            toffs = tpar[pl.ds(c * CHUNK + gbase, LANES)]
            soffs = spar[pl.ds(c * CHUNK + gbase, LANES)]
            res = jnp.zeros((LANES,), jnp.float32)
            for j in range(LANES):
                row = gbase + j
                toff = toffs[j]
                soff = soffs[j]
                acc = jnp.zeros((LANES,), jnp.float32)
                for k in range(n_slices):
                    tsl2 = pl.ds(toff + k * LANES, LANES)
                    ssl2 = pl.ds(soff + k * LANES, LANES)
                    grad = wt[row, tsl2] - ws[row, ssl2]
                    gsum = gt[row, tsl2] + gs[row, ssl2]
                    denom = 1.0 + jnp.exp(-gsum)
                    acc = acc + grad / denom
                s = jnp.sum(acc)
                res = jnp.where(lane == j, s, res)
            out_v[pl.ds(c * CHUNK + g * LANES, LANES)] = res
            return 0

        lax.fori_loop(0, CHUNK // LANES, group_body, 0)

    pltpu.sync_copy(out_v, out_hbm.at[pl.ds(base, b_per_w)])


@jax.jit
def kernel(source_nodes, target_nodes, node_embeddings, gates):
    batch = source_nodes.shape[0]
    num_nodes = node_embeddings.shape[0]
    b_per_w = batch // NW
    emb2 = node_embeddings.reshape(num_nodes // 2, PAIR)
    gat2 = gates.reshape(num_nodes // 2, PAIR)
    mesh = plsc.VectorSubcoreMesh(
        core_axis_name="c", subcore_axis_name="s",
        num_cores=NUM_CORES, num_subcores=NUM_SUBCORES)
    run = pl.kernel(
        functools.partial(_body, b_per_w=b_per_w),
        out_type=jax.ShapeDtypeStruct((batch,), jnp.float32),
        mesh=mesh,
        compiler_params=pltpu.CompilerParams(needs_layout_passes=False),
        scratch_types=[
            pltpu.VMEM((b_per_w,), jnp.int32),
            pltpu.VMEM((b_per_w,), jnp.int32),
            pltpu.VMEM((b_per_w,), jnp.int32),
            pltpu.VMEM((b_per_w,), jnp.int32),
            pltpu.VMEM((CHUNK, PAIR), jnp.float32),
            pltpu.VMEM((CHUNK, PAIR), jnp.float32),
            pltpu.VMEM((CHUNK, PAIR), jnp.float32),
            pltpu.VMEM((CHUNK, PAIR), jnp.float32),
            pltpu.VMEM((b_per_w,), jnp.float32),
            pltpu.SemaphoreType.DMA,
        ],
    )
    return run(source_nodes.astype(jnp.int32), target_nodes.astype(jnp.int32),
               emb2, gat2)


# speedup vs baseline: 2.2717x; 2.2690x over previous
import functools

import jax
import jax.numpy as jnp
from jax import lax
from jax.experimental import pallas as pl
from jax.experimental.pallas import tpu as pltpu
from jax.experimental.pallas import tpu_sc as plsc

NUM_CORES = 2
NUM_SUBCORES = 16
LANES = 16
NW = NUM_CORES * NUM_SUBCORES

EMBED_DIM = 64
ROW = 2 * EMBED_DIM
TILE_W = 128
SCAT = 128


def _gather_body(sn_hbm, ord_hbm, emb_hbm, gat_hbm, scr_hbm,
                 sn_v, ord_v, embc, gatc, stage, sem, *, i_per_w):
    wid = lax.axis_index("s") * NUM_CORES + lax.axis_index("c")
    rbase = wid * (i_per_w // TILE_W)

    pltpu.sync_copy(sn_hbm.at[pl.ds(rbase, i_per_w // TILE_W)], sn_v)
    pltpu.sync_copy(ord_hbm.at[pl.ds(rbase, i_per_w // TILE_W)], ord_v)

    lane = lax.iota(jnp.int32, LANES)
    n_groups = i_per_w // LANES
    gpr = TILE_W // LANES

    def grp(g, cur_tc):
        vrow = g // gpr
        voff = (g % gpr) * LANES
        sv = sn_v[vrow, pl.ds(voff, LANES)]
        tcs = lax.shift_right_logical(sv, 7)
        lns = lax.bitwise_and(sv, 127)
        for j in range(LANES):
            tc_j = tcs[j]
            l_j = lns[j]

            @pl.when(tc_j != cur_tc)
            def _():
                a = pltpu.async_copy(
                    emb_hbm.at[:, pl.ds(tc_j * TILE_W, TILE_W)], embc, sem)
                b = pltpu.async_copy(
                    gat_hbm.at[:, pl.ds(tc_j * TILE_W, TILE_W)], gatc, sem)
                a.wait()
                b.wait()

            srow = (g % (SCAT // LANES)) * LANES + j
            lvec = jnp.full((LANES,), 0, jnp.int32) + l_j
            for k in range(EMBED_DIM // LANES):
                idx_d = lane + (k * LANES)
                ev = plsc.load_gather(embc, [idx_d, lvec])
                gv = plsc.load_gather(gatc, [idx_d, lvec])
                stage[srow, pl.ds(k * LANES, LANES)] = ev
                stage[srow, pl.ds(EMBED_DIM + k * LANES, LANES)] = gv
            cur_tc = tc_j

        @pl.when(g % (SCAT // LANES) == (SCAT // LANES) - 1)
        def _():
            q = g // (SCAT // LANES)
            pltpu.async_copy(
                stage, scr_hbm.at[ord_v.at[q]], sem).wait()

        return cur_tc

    lax.fori_loop(0, n_groups, grp, jnp.int32(-1))


def _combine_body(scr_hbm, out_hbm, rt, rs, out_v, sem, *, b_per_w, batch):
    wid = lax.axis_index("s") * NUM_CORES + lax.axis_index("c")
    base = wid * b_per_w

    lane = lax.iota(jnp.int32, LANES)
    chunk = rt.shape[0]
    n_chunks = b_per_w // chunk
    n_slices = EMBED_DIM // LANES
    n_groups = chunk // LANES

    for c in range(n_chunks):
        cps = [
            pltpu.async_copy(
                scr_hbm.at[pl.ds(base + c * chunk, chunk)], rt, sem),
            pltpu.async_copy(
                scr_hbm.at[pl.ds(batch + base + c * chunk, chunk)], rs, sem),
        ]
        for cp in cps:
            cp.wait()

        def group_body(g, _, c=c):
            res = jnp.zeros((LANES,), jnp.float32)
            for j in range(LANES):
                row = g * LANES + j
                acc = jnp.zeros((LANES,), jnp.float32)
                for k in range(n_slices):
                    esl = pl.ds(k * LANES, LANES)
                    gsl = pl.ds(EMBED_DIM + k * LANES, LANES)
                    grad = rt[row, esl] - rs[row, esl]
                    gsum = rt[row, gsl] + rs[row, gsl]
                    denom = 1.0 + jnp.exp(-gsum)
                    acc = acc + grad / denom
                s = jnp.sum(acc)
                res = jnp.where(lane == j, s, res)
            out_v[pl.ds(c * chunk + g * LANES, LANES)] = res
            return 0

        lax.fori_loop(0, n_groups, group_body, 0)

    pltpu.sync_copy(out_v, out_hbm.at[pl.ds(base, b_per_w)])


@jax.jit
def kernel(source_nodes, target_nodes, node_embeddings, gates):
    batch = source_nodes.shape[0]
    n_items = 2 * batch
    i_per_w = n_items // NW
    b_per_w = batch // NW
    chunk = 128

    nodes_all = jnp.concatenate(
        [target_nodes.astype(jnp.int32), source_nodes.astype(jnp.int32)])
    order = jnp.argsort(nodes_all).astype(jnp.int32)
    snodes = jnp.take(nodes_all, order)
    sn2 = snodes.reshape(n_items // TILE_W, TILE_W)
    ord2 = order.reshape(n_items // TILE_W, TILE_W)

    mesh = plsc.VectorSubcoreMesh(
        core_axis_name="c", subcore_axis_name="s",
        num_cores=NUM_CORES, num_subcores=NUM_SUBCORES)
    params = pltpu.CompilerParams(needs_layout_passes=False)

    gather = pl.kernel(
        functools.partial(_gather_body, i_per_w=i_per_w),
        out_type=jax.ShapeDtypeStruct((n_items, ROW), jnp.float32),
        mesh=mesh,
        compiler_params=params,
        scratch_types=[
            pltpu.VMEM((i_per_w // TILE_W, TILE_W), jnp.int32),
            pltpu.VMEM((i_per_w // TILE_W, TILE_W), jnp.int32),
            pltpu.VMEM((EMBED_DIM, TILE_W), jnp.float32),
            pltpu.VMEM((EMBED_DIM, TILE_W), jnp.float32),
            pltpu.VMEM((SCAT, ROW), jnp.float32),
            pltpu.SemaphoreType.DMA,
        ],
    )
    scratch = gather(sn2, ord2, node_embeddings.T, gates.T)

    combine = pl.kernel(
        functools.partial(_combine_body, b_per_w=b_per_w, batch=batch),
        out_type=jax.ShapeDtypeStruct((batch,), jnp.float32),
        mesh=mesh,
        compiler_params=params,
        scratch_types=[
            pltpu.VMEM((chunk, ROW), jnp.float32),
            pltpu.VMEM((chunk, ROW), jnp.float32),
            pltpu.VMEM((b_per_w,), jnp.float32),
            pltpu.SemaphoreType.DMA,
        ],
    )
    return combine(scratch)


# speedup vs baseline: 2.9226x; 1.2865x over previous
import functools

import jax
import jax.numpy as jnp
from jax import lax
from jax.experimental import pallas as pl
from jax.experimental.pallas import tpu as pltpu
from jax.experimental.pallas import tpu_sc as plsc

NUM_CORES = 2
NUM_SUBCORES = 16
LANES = 16
NW = NUM_CORES * NUM_SUBCORES

EMBED_DIM = 64
ROW = 2 * EMBED_DIM
TILE_W = 128
SCAT = 128
RING = 4
CSEQ = 1056


def _col_at(cseq, q):
    grp = cseq[pl.ds((q // LANES) * LANES, LANES)]
    sel = lax.iota(jnp.int32, LANES) == (q % LANES)
    return jnp.sum(jnp.where(sel, grp, 0))


def _fetch(emb_hbm, gat_hbm, embr, gatr, sems, slot, col):
    rmod = slot % RING
    sl = pl.ds(col * TILE_W, TILE_W)
    pltpu.async_copy(emb_hbm.at[:, sl], embr.at[rmod], sems.at[rmod])
    pltpu.async_copy(gat_hbm.at[:, sl], gatr.at[rmod], sems.at[rmod])


def _await(emb_hbm, gat_hbm, embr, gatr, sems, slot):
    rmod = slot % RING
    pltpu.make_async_copy(
        emb_hbm.at[:, pl.ds(0, TILE_W)], embr.at[rmod], sems.at[rmod]).wait()
    pltpu.make_async_copy(
        gat_hbm.at[:, pl.ds(0, TILE_W)], gatr.at[rmod], sems.at[rmod]).wait()


def _gather_body(sn_hbm, ord_hbm, seg_hbm, cseq_hbm, emb_hbm, gat_hbm,
                 scr_hbm, sn_v, ord_v, seg_v, cseq_v, embr, gatr, stage,
                 sems, ssem, *, i_per_w):
    wid = lax.axis_index("s") * NUM_CORES + lax.axis_index("c")
    vrows = i_per_w // TILE_W
    rbase = wid * vrows

    pltpu.sync_copy(sn_hbm.at[pl.ds(rbase, vrows)], sn_v)
    pltpu.sync_copy(ord_hbm.at[pl.ds(rbase, vrows)], ord_v)
    pltpu.sync_copy(seg_hbm.at[pl.ds(rbase, vrows)], seg_v)

    s0 = jnp.sum(jnp.where(lax.iota(jnp.int32, LANES) == 0,
                           seg_v[0, pl.ds(0, LANES)], 0))
    s0a = (s0 // LANES) * LANES
    pltpu.sync_copy(cseq_hbm.at[pl.ds(s0a, CSEQ)], cseq_v)

    lane = lax.iota(jnp.int32, LANES)
    n_groups = i_per_w // LANES
    gpr = TILE_W // LANES

    def prime(r, _):
        _fetch(emb_hbm, gat_hbm, embr, gatr, sems, s0 + r,
               _col_at(cseq_v, s0 + r - s0a))
        return 0
    lax.fori_loop(0, RING, prime, 0)
    _await(emb_hbm, gat_hbm, embr, gatr, sems, s0)

    def grp(g, cur):
        vrow = g // gpr
        voff = (g % gpr) * LANES
        segs = seg_v[vrow, pl.ds(voff, LANES)]
        sv = sn_v[vrow, pl.ds(voff, LANES)]
        lns = lax.bitwise_and(sv, 127)
        srow0 = (g % (SCAT // LANES)) * LANES
        for j in range(LANES):
            s_j = segs[j]
            l_j = lns[j]

            @pl.when(s_j != cur)
            def _():
                _fetch(emb_hbm, gat_hbm, embr, gatr, sems, cur,
                       _col_at(cseq_v, cur + RING - s0a))
                _await(emb_hbm, gat_hbm, embr, gatr, sems, s_j)

            cur = s_j
            rmod = cur % RING
            rvec = jnp.full((LANES,), 0, jnp.int32) + rmod
            lvec = jnp.full((LANES,), 0, jnp.int32) + l_j
            for k in range(EMBED_DIM // LANES):
                idx_d = lane + (k * LANES)
                ev = plsc.load_gather(embr, [rvec, idx_d, lvec])
                gv = plsc.load_gather(gatr, [rvec, idx_d, lvec])
                stage[srow0 + j, pl.ds(k * LANES, LANES)] = ev
                stage[srow0 + j, pl.ds(EMBED_DIM + k * LANES, LANES)] = gv

        @pl.when(g % (SCAT // LANES) == (SCAT // LANES) - 1)
        def _():
            q = g // (SCAT // LANES)
            pltpu.async_copy(stage, scr_hbm.at[ord_v.at[q]], ssem).wait()

        return cur

    cur = lax.fori_loop(0, n_groups, grp, s0)

    def drain(r, _):
        _await(emb_hbm, gat_hbm, embr, gatr, sems, cur + r)
        return 0
    lax.fori_loop(1, RING, drain, 0)


def _combine_body(scr_hbm, out_hbm, rt, rs, out_v, sem, *, b_per_w, batch):
    wid = lax.axis_index("s") * NUM_CORES + lax.axis_index("c")
    base = wid * b_per_w

    lane = lax.iota(jnp.int32, LANES)
    chunk = rt.shape[0]
    n_chunks = b_per_w // chunk
    n_slices = EMBED_DIM // LANES
    n_groups = chunk // LANES

    for c in range(n_chunks):
        cps = [
            pltpu.async_copy(
                scr_hbm.at[pl.ds(base + c * chunk, chunk)], rt, sem),
            pltpu.async_copy(
                scr_hbm.at[pl.ds(batch + base + c * chunk, chunk)], rs, sem),
        ]
        for cp in cps:
            cp.wait()

        def group_body(g, _, c=c):
            res = jnp.zeros((LANES,), jnp.float32)
            for j in range(LANES):
                row = g * LANES + j
                acc = jnp.zeros((LANES,), jnp.float32)
                for k in range(n_slices):
                    esl = pl.ds(k * LANES, LANES)
                    gsl = pl.ds(EMBED_DIM + k * LANES, LANES)
                    grad = rt[row, esl] - rs[row, esl]
                    gsum = rt[row, gsl] + rs[row, gsl]
                    denom = 1.0 + jnp.exp(-gsum)
                    acc = acc + grad / denom
                s = jnp.sum(acc)
                res = jnp.where(lane == j, s, res)
            out_v[pl.ds(c * chunk + g * LANES, LANES)] = res
            return 0

        lax.fori_loop(0, n_groups, group_body, 0)

    pltpu.sync_copy(out_v, out_hbm.at[pl.ds(base, b_per_w)])


@jax.jit
def kernel(source_nodes, target_nodes, node_embeddings, gates):
    batch = source_nodes.shape[0]
    n_items = 2 * batch
    i_per_w = n_items // NW
    b_per_w = batch // NW
    chunk = 128

    nodes_all = jnp.concatenate(
        [target_nodes.astype(jnp.int32), source_nodes.astype(jnp.int32)])
    order = jnp.argsort(nodes_all).astype(jnp.int32)
    snodes = jnp.take(nodes_all, order)
    tcs = lax.shift_right_logical(snodes, 7)
    newcol = jnp.concatenate(
        [jnp.ones((1,), jnp.int32),
         (tcs[1:] != tcs[:-1]).astype(jnp.int32)])
    seg = jnp.cumsum(newcol, dtype=jnp.int32) - 1
    cseq = jnp.zeros((n_items + 2 * CSEQ,), jnp.int32).at[seg].set(tcs)

    sn2 = snodes.reshape(n_items // TILE_W, TILE_W)
    ord2 = order.reshape(n_items // TILE_W, TILE_W)
    seg2 = seg.reshape(n_items // TILE_W, TILE_W)

    mesh = plsc.VectorSubcoreMesh(
        core_axis_name="c", subcore_axis_name="s",
        num_cores=NUM_CORES, num_subcores=NUM_SUBCORES)
    params = pltpu.CompilerParams(needs_layout_passes=False)

    gather = pl.kernel(
        functools.partial(_gather_body, i_per_w=i_per_w),
        out_type=jax.ShapeDtypeStruct((n_items, ROW), jnp.float32),
        mesh=mesh,
        compiler_params=params,
        scratch_types=[
            pltpu.VMEM((i_per_w // TILE_W, TILE_W), jnp.int32),
            pltpu.VMEM((i_per_w // TILE_W, TILE_W), jnp.int32),
            pltpu.VMEM((i_per_w // TILE_W, TILE_W), jnp.int32),
            pltpu.VMEM((CSEQ,), jnp.int32),
            pltpu.VMEM((RING, EMBED_DIM, TILE_W), jnp.float32),
            pltpu.VMEM((RING, EMBED_DIM, TILE_W), jnp.float32),
            pltpu.VMEM((SCAT, ROW), jnp.float32),
            pltpu.SemaphoreType.DMA((RING,)),
            pltpu.SemaphoreType.DMA,
        ],
    )
    scratch = gather(sn2, ord2, seg2, cseq, node_embeddings.T, gates.T)

    combine = pl.kernel(
        functools.partial(_combine_body, b_per_w=b_per_w, batch=batch),
        out_type=jax.ShapeDtypeStruct((batch,), jnp.float32),
        mesh=mesh,
        compiler_params=params,
        scratch_types=[
            pltpu.VMEM((chunk, ROW), jnp.float32),
            pltpu.VMEM((chunk, ROW), jnp.float32),
            pltpu.VMEM((b_per_w,), jnp.float32),
            pltpu.SemaphoreType.DMA,
        ],
    )
    return combine(scratch)


# speedup vs baseline: 3.9265x; 1.3435x over previous
import functools

import jax
import jax.numpy as jnp
from jax import lax
from jax.experimental import pallas as pl
from jax.experimental.pallas import tpu as pltpu
from jax.experimental.pallas import tpu_sc as plsc

NUM_CORES = 2
NUM_SUBCORES = 16
LANES = 16
NW = NUM_CORES * NUM_SUBCORES

EMBED_DIM = 64
ROW = 2 * EMBED_DIM
TILE_W = 128
SCAT = 128
RING = 4
CSEQ = 1056


def _col_at(cseq, q):
    grp = cseq[pl.ds((q // LANES) * LANES, LANES)]
    sel = lax.iota(jnp.int32, LANES) == (q % LANES)
    return jnp.sum(jnp.where(sel, grp, 0))


def _fetch(emb_hbm, gat_hbm, embr, gatr, sems, slot, col):
    rmod = slot % RING
    sl = pl.ds(col * TILE_W, TILE_W)
    pltpu.async_copy(emb_hbm.at[:, sl], embr.at[rmod], sems.at[rmod])
    pltpu.async_copy(gat_hbm.at[:, sl], gatr.at[rmod], sems.at[rmod])


def _await(emb_hbm, gat_hbm, embr, gatr, sems, slot):
    rmod = slot % RING
    pltpu.make_async_copy(
        emb_hbm.at[:, pl.ds(0, TILE_W)], embr.at[rmod], sems.at[rmod]).wait()
    pltpu.make_async_copy(
        gat_hbm.at[:, pl.ds(0, TILE_W)], gatr.at[rmod], sems.at[rmod]).wait()


def _gather_body(sn_hbm, ord_hbm, seg_hbm, cseq_hbm, emb_hbm, gat_hbm,
                 scr_hbm, sn_v, ord_v, seg_v, cseq_v, embr, gatr, stage,
                 sems, ssem, *, i_per_w):
    wid = lax.axis_index("s") * NUM_CORES + lax.axis_index("c")
    vrows = i_per_w // TILE_W
    rbase = wid * vrows

    pltpu.sync_copy(sn_hbm.at[pl.ds(rbase, vrows)], sn_v)
    pltpu.sync_copy(ord_hbm.at[pl.ds(rbase, vrows)], ord_v)
    pltpu.sync_copy(seg_hbm.at[pl.ds(rbase, vrows)], seg_v)

    s0 = jnp.sum(jnp.where(lax.iota(jnp.int32, LANES) == 0,
                           seg_v[0, pl.ds(0, LANES)], 0))
    s0a = (s0 // LANES) * LANES
    pltpu.sync_copy(cseq_hbm.at[pl.ds(s0a, CSEQ)], cseq_v)

    lane = lax.iota(jnp.int32, LANES)
    n_groups = i_per_w // LANES
    gpr = TILE_W // LANES

    def prime(r, _):
        _fetch(emb_hbm, gat_hbm, embr, gatr, sems, s0 + r,
               _col_at(cseq_v, s0 + r - s0a))
        return 0
    lax.fori_loop(0, RING, prime, 0)
    _await(emb_hbm, gat_hbm, embr, gatr, sems, s0)

    def grp(g, cur):
        vrow = g // gpr
        voff = (g % gpr) * LANES
        segs = seg_v[vrow, pl.ds(voff, LANES)]
        sv = sn_v[vrow, pl.ds(voff, LANES)]
        lns = lax.bitwise_and(sv, 127)
        srow0 = (g % (SCAT // LANES)) * LANES
        for j in range(LANES):
            s_j = segs[j]
            l_j = lns[j]

            @pl.when(s_j != cur)
            def _():
                _fetch(emb_hbm, gat_hbm, embr, gatr, sems, cur,
                       _col_at(cseq_v, cur + RING - s0a))
                _await(emb_hbm, gat_hbm, embr, gatr, sems, s_j)

            cur = s_j
            rmod = cur % RING
            rvec = jnp.full((LANES,), 0, jnp.int32) + rmod
            lvec = jnp.full((LANES,), 0, jnp.int32) + l_j
            for k in range(EMBED_DIM // LANES):
                idx_d = lane + (k * LANES)
                ev = plsc.load_gather(embr, [rvec, idx_d, lvec])
                gv = plsc.load_gather(gatr, [rvec, idx_d, lvec])
                stage[srow0 + j, pl.ds(k * LANES, LANES)] = ev
                stage[srow0 + j, pl.ds(EMBED_DIM + k * LANES, LANES)] = gv

        @pl.when(g % (SCAT // LANES) == (SCAT // LANES) - 1)
        def _():
            q = g // (SCAT // LANES)
            pltpu.async_copy(stage, scr_hbm.at[ord_v.at[q]], ssem).wait()

        return cur

    cur = lax.fori_loop(0, n_groups, grp, s0)

    def drain(r, _):
        _await(emb_hbm, gat_hbm, embr, gatr, sems, cur + r)
        return 0
    lax.fori_loop(1, RING, drain, 0)


def _combine_body(scr_hbm, out_hbm, rt, rs, out_v, sem, *, b_per_w, batch):
    wid = lax.axis_index("s") * NUM_CORES + lax.axis_index("c")
    base = wid * b_per_w

    lane = lax.iota(jnp.int32, LANES)
    chunk = rt.shape[0]
    n_chunks = b_per_w // chunk
    n_slices = EMBED_DIM // LANES
    n_groups = chunk // LANES

    for c in range(n_chunks):
        cps = [
            pltpu.async_copy(
                scr_hbm.at[pl.ds(base + c * chunk, chunk)], rt, sem),
            pltpu.async_copy(
                scr_hbm.at[pl.ds(batch + base + c * chunk, chunk)], rs, sem),
        ]
        for cp in cps:
            cp.wait()

        def group_body(g, _, c=c):
            res = jnp.zeros((LANES,), jnp.float32)
            for j in range(LANES):
                row = g * LANES + j
                acc = jnp.zeros((LANES,), jnp.float32)
                for k in range(n_slices):
                    esl = pl.ds(k * LANES, LANES)
                    gsl = pl.ds(EMBED_DIM + k * LANES, LANES)
                    grad = rt[row, esl] - rs[row, esl]
                    gsum = rt[row, gsl] + rs[row, gsl]
                    denom = 1.0 + jnp.exp(-gsum)
                    acc = acc + grad / denom
                s = jnp.sum(acc)
                res = jnp.where(lane == j, s, res)
            out_v[pl.ds(c * chunk + g * LANES, LANES)] = res
            return 0

        lax.fori_loop(0, n_groups, group_body, 0)

    pltpu.sync_copy(out_v, out_hbm.at[pl.ds(base, b_per_w)])


@jax.jit
def kernel(source_nodes, target_nodes, node_embeddings, gates):
    batch = source_nodes.shape[0]
    n_items = 2 * batch
    i_per_w = n_items // NW
    b_per_w = batch // NW
    chunk = 128

    nodes_all = jnp.concatenate(
        [target_nodes.astype(jnp.int32), source_nodes.astype(jnp.int32)])
    order = jnp.argsort(nodes_all).astype(jnp.int32)
    snodes = jnp.take(nodes_all, order)
    tcs = lax.shift_right_logical(snodes, 7)
    newcol = jnp.concatenate(
        [jnp.ones((1,), jnp.int32),
         (tcs[1:] != tcs[:-1]).astype(jnp.int32)])
    seg = jnp.cumsum(newcol, dtype=jnp.int32) - 1
    heads = jnp.argsort(1 - newcol, stable=True).astype(jnp.int32)
    cseq = jnp.concatenate(
        [jnp.take(tcs, heads), jnp.zeros((2 * CSEQ,), jnp.int32)])

    sn2 = snodes.reshape(n_items // TILE_W, TILE_W)
    ord2 = order.reshape(n_items // TILE_W, TILE_W)
    seg2 = seg.reshape(n_items // TILE_W, TILE_W)

    mesh = plsc.VectorSubcoreMesh(
        core_axis_name="c", subcore_axis_name="s",
        num_cores=NUM_CORES, num_subcores=NUM_SUBCORES)
    params = pltpu.CompilerParams(needs_layout_passes=False)

    gather = pl.kernel(
        functools.partial(_gather_body, i_per_w=i_per_w),
        out_type=jax.ShapeDtypeStruct((n_items, ROW), jnp.float32),
        mesh=mesh,
        compiler_params=params,
        scratch_types=[
            pltpu.VMEM((i_per_w // TILE_W, TILE_W), jnp.int32),
            pltpu.VMEM((i_per_w // TILE_W, TILE_W), jnp.int32),
            pltpu.VMEM((i_per_w // TILE_W, TILE_W), jnp.int32),
            pltpu.VMEM((CSEQ,), jnp.int32),
            pltpu.VMEM((RING, EMBED_DIM, TILE_W), jnp.float32),
            pltpu.VMEM((RING, EMBED_DIM, TILE_W), jnp.float32),
            pltpu.VMEM((SCAT, ROW), jnp.float32),
            pltpu.SemaphoreType.DMA((RING,)),
            pltpu.SemaphoreType.DMA,
        ],
    )
    scratch = gather(sn2, ord2, seg2, cseq, node_embeddings.T, gates.T)

    combine = pl.kernel(
        functools.partial(_combine_body, b_per_w=b_per_w, batch=batch),
        out_type=jax.ShapeDtypeStruct((batch,), jnp.float32),
        mesh=mesh,
        compiler_params=params,
        scratch_types=[
            pltpu.VMEM((chunk, ROW), jnp.float32),
            pltpu.VMEM((chunk, ROW), jnp.float32),
            pltpu.VMEM((b_per_w,), jnp.float32),
            pltpu.SemaphoreType.DMA,
        ],
    )
    return combine(scratch)


# speedup vs baseline: 4.2184x; 1.0743x over previous
import functools

import jax
import jax.numpy as jnp
from jax import lax
from jax.experimental import pallas as pl
from jax.experimental.pallas import tpu as pltpu
from jax.experimental.pallas import tpu_sc as plsc

NUM_CORES = 2
NUM_SUBCORES = 16
LANES = 16
NW = NUM_CORES * NUM_SUBCORES

EMBED_DIM = 64
ROW = 2 * EMBED_DIM
TILE_W = 128
SCAT = 128
RING = 6
CSEQ = 1056


def _col_at(cseq, q):
    grp = cseq[pl.ds((q // LANES) * LANES, LANES)]
    sel = lax.iota(jnp.int32, LANES) == (q % LANES)
    return jnp.sum(jnp.where(sel, grp, 0))


def _fetch(emb_hbm, gat_hbm, embr, gatr, sems, slot, col):
    rmod = slot % RING
    sl = pl.ds(col * TILE_W, TILE_W)
    pltpu.async_copy(emb_hbm.at[:, sl], embr.at[rmod], sems.at[rmod])
    pltpu.async_copy(gat_hbm.at[:, sl], gatr.at[rmod], sems.at[rmod])


def _await(emb_hbm, gat_hbm, embr, gatr, sems, slot):
    rmod = slot % RING
    pltpu.make_async_copy(
        emb_hbm.at[:, pl.ds(0, TILE_W)], embr.at[rmod], sems.at[rmod]).wait()
    pltpu.make_async_copy(
        gat_hbm.at[:, pl.ds(0, TILE_W)], gatr.at[rmod], sems.at[rmod]).wait()


def _gather_body(sn_hbm, ord_hbm, seg_hbm, cseq_hbm, emb_hbm, gat_hbm,
                 scr_hbm, sn_v, ord_v, seg_v, cseq_v, embr, gatr, stage,
                 sems, ssem, *, i_per_w):
    wid = lax.axis_index("s") * NUM_CORES + lax.axis_index("c")
    vrows = i_per_w // TILE_W
    rbase = wid * vrows

    pltpu.sync_copy(sn_hbm.at[pl.ds(rbase, vrows)], sn_v)
    pltpu.sync_copy(ord_hbm.at[pl.ds(rbase, vrows)], ord_v)
    pltpu.sync_copy(seg_hbm.at[pl.ds(rbase, vrows)], seg_v)

    s0 = jnp.sum(jnp.where(lax.iota(jnp.int32, LANES) == 0,
                           seg_v[0, pl.ds(0, LANES)], 0))
    s0a = (s0 // LANES) * LANES
    pltpu.sync_copy(cseq_hbm.at[pl.ds(s0a, CSEQ)], cseq_v)

    lane = lax.iota(jnp.int32, LANES)
    n_groups = i_per_w // LANES
    gpr = TILE_W // LANES

    def prime(r, _):
        _fetch(emb_hbm, gat_hbm, embr, gatr, sems, s0 + r,
               _col_at(cseq_v, s0 + r - s0a))
        return 0
    lax.fori_loop(0, RING, prime, 0)
    _await(emb_hbm, gat_hbm, embr, gatr, sems, s0)

    def grp(g, cur):
        vrow = g // gpr
        voff = (g % gpr) * LANES
        segs = seg_v[vrow, pl.ds(voff, LANES)]
        sv = sn_v[vrow, pl.ds(voff, LANES)]
        lns = lax.bitwise_and(sv, 127)
        srow0 = (g % (SCAT // LANES)) * LANES
        for j in range(LANES):
            s_j = segs[j]
            l_j = lns[j]

            @pl.when(s_j != cur)
            def _():
                _fetch(emb_hbm, gat_hbm, embr, gatr, sems, cur,
                       _col_at(cseq_v, cur + RING - s0a))
                _await(emb_hbm, gat_hbm, embr, gatr, sems, s_j)

            cur = s_j
            rmod = cur % RING
            rvec = jnp.full((LANES,), 0, jnp.int32) + rmod
            lvec = jnp.full((LANES,), 0, jnp.int32) + l_j
            for k in range(EMBED_DIM // LANES):
                idx_d = lane + (k * LANES)
                ev = plsc.load_gather(embr, [rvec, idx_d, lvec])
                gv = plsc.load_gather(gatr, [rvec, idx_d, lvec])
                stage[srow0 + j, pl.ds(k * LANES, LANES)] = ev
                stage[srow0 + j, pl.ds(EMBED_DIM + k * LANES, LANES)] = gv

        @pl.when(g % (SCAT // LANES) == (SCAT // LANES) - 1)
        def _():
            q = g // (SCAT // LANES)
            pltpu.async_copy(stage, scr_hbm.at[ord_v.at[q]], ssem).wait()

        return cur

    cur = lax.fori_loop(0, n_groups, grp, s0)

    def drain(r, _):
        _await(emb_hbm, gat_hbm, embr, gatr, sems, cur + r)
        return 0
    lax.fori_loop(1, RING, drain, 0)


def _combine_body(scr_hbm, out_hbm, rt2, rs2, out_v, sems, *, b_per_w, batch):
    wid = lax.axis_index("s") * NUM_CORES + lax.axis_index("c")
    base = wid * b_per_w

    lane = lax.iota(jnp.int32, LANES)
    chunk = rt2.shape[1]
    n_chunks = b_per_w // chunk
    n_slices = EMBED_DIM // LANES
    n_groups = chunk // LANES

    def issue(c):
        par = c % 2
        pltpu.async_copy(
            scr_hbm.at[pl.ds(base + c * chunk, chunk)], rt2.at[par],
            sems.at[par])
        pltpu.async_copy(
            scr_hbm.at[pl.ds(batch + base + c * chunk, chunk)], rs2.at[par],
            sems.at[par])

    issue(0)
    for c in range(n_chunks):
        par = c % 2
        if c + 1 < n_chunks:
            issue(c + 1)
        pltpu.make_async_copy(
            scr_hbm.at[pl.ds(0, chunk)], rt2.at[par], sems.at[par]).wait()
        pltpu.make_async_copy(
            scr_hbm.at[pl.ds(0, chunk)], rs2.at[par], sems.at[par]).wait()

        def group_body(g, _, c=c, par=par):
            res = jnp.zeros((LANES,), jnp.float32)
            for j in range(LANES):
                row = g * LANES + j
                acc = jnp.zeros((LANES,), jnp.float32)
                for k in range(n_slices):
                    esl = pl.ds(k * LANES, LANES)
                    gsl = pl.ds(EMBED_DIM + k * LANES, LANES)
                    grad = rt2[par, row, esl] - rs2[par, row, esl]
                    gsum = rt2[par, row, gsl] + rs2[par, row, gsl]
                    denom = 1.0 + jnp.exp(-gsum)
                    acc = acc + grad / denom
                s = jnp.sum(acc)
                res = jnp.where(lane == j, s, res)
            out_v[pl.ds(c * chunk + g * LANES, LANES)] = res
            return 0

        lax.fori_loop(0, n_groups, group_body, 0)

    pltpu.sync_copy(out_v, out_hbm.at[pl.ds(base, b_per_w)])


@jax.jit
def kernel(source_nodes, target_nodes, node_embeddings, gates):
    batch = source_nodes.shape[0]
    n_items = 2 * batch
    i_per_w = n_items // NW
    b_per_w = batch // NW
    chunk = 128

    nodes_all = jnp.concatenate(
        [target_nodes.astype(jnp.int32), source_nodes.astype(jnp.int32)])
    order = jnp.argsort(nodes_all).astype(jnp.int32)
    snodes = jnp.take(nodes_all, order)
    tcs = lax.shift_right_logical(snodes, 7)
    newcol = jnp.concatenate(
        [jnp.ones((1,), jnp.int32),
         (tcs[1:] != tcs[:-1]).astype(jnp.int32)])
    seg = jnp.cumsum(newcol, dtype=jnp.int32) - 1
    heads = jnp.argsort(1 - newcol, stable=True).astype(jnp.int32)
    cseq = jnp.concatenate(
        [jnp.take(tcs, heads), jnp.zeros((2 * CSEQ,), jnp.int32)])

    sn2 = snodes.reshape(n_items // TILE_W, TILE_W)
    ord2 = order.reshape(n_items // TILE_W, TILE_W)
    seg2 = seg.reshape(n_items // TILE_W, TILE_W)

    mesh = plsc.VectorSubcoreMesh(
        core_axis_name="c", subcore_axis_name="s",
        num_cores=NUM_CORES, num_subcores=NUM_SUBCORES)
    params = pltpu.CompilerParams(needs_layout_passes=False)

    gather = pl.kernel(
        functools.partial(_gather_body, i_per_w=i_per_w),
        out_type=jax.ShapeDtypeStruct((n_items, ROW), jnp.float32),
        mesh=mesh,
        compiler_params=params,
        scratch_types=[
            pltpu.VMEM((i_per_w // TILE_W, TILE_W), jnp.int32),
            pltpu.VMEM((i_per_w // TILE_W, TILE_W), jnp.int32),
            pltpu.VMEM((i_per_w // TILE_W, TILE_W), jnp.int32),
            pltpu.VMEM((CSEQ,), jnp.int32),
            pltpu.VMEM((RING, EMBED_DIM, TILE_W), jnp.float32),
            pltpu.VMEM((RING, EMBED_DIM, TILE_W), jnp.float32),
            pltpu.VMEM((SCAT, ROW), jnp.float32),
            pltpu.SemaphoreType.DMA((RING,)),
            pltpu.SemaphoreType.DMA,
        ],
    )
    scratch = gather(sn2, ord2, seg2, cseq, node_embeddings.T, gates.T)

    combine = pl.kernel(
        functools.partial(_combine_body, b_per_w=b_per_w, batch=batch),
        out_type=jax.ShapeDtypeStruct((batch,), jnp.float32),
        mesh=mesh,
        compiler_params=params,
        scratch_types=[
            pltpu.VMEM((2, chunk, ROW), jnp.float32),
            pltpu.VMEM((2, chunk, ROW), jnp.float32),
            pltpu.VMEM((b_per_w,), jnp.float32),
            pltpu.SemaphoreType.DMA((2,)),
        ],
    )
    return combine(scratch)
